# Initial kernel scaffold; baseline (speedup 1.0000x reference)
import jax
import jax.numpy as jnp
from jax.experimental import pallas as pl


def kernel(xyz, xyz_s, fea, fea_s, Wf, bf, Wg, bg):
    raise NotImplementedError("write your pallas kernel here")


# speedup vs baseline: 15.2315x; 15.2315x over previous
import functools

import jax
import jax.numpy as jnp
from jax import lax
from jax.experimental import pallas as pl
from jax.experimental.pallas import tpu as pltpu
from jax.experimental.pallas import tpu_sc as plsc

KNN = 16
TW = 256
NEG_INF = float("-inf")

SC_NC = 2
SC_NS = 16
SC_NW = SC_NC * SC_NS


def _prep_body(fea_ref, fea_s_ref, xyzT_ref, xyz_sT_ref, wf1_ref, wf2_ref,
               wge_ref, bf_ref, bg_ref, t_ref, ap_ref):
    fea = fea_ref[0]
    fea_s = fea_s_ref[0]
    xq = xyzT_ref[0]
    xs = xyz_sT_ref[0]
    dn = (((0,), (0,)), ((), ()))
    a = lax.dot_general(fea, wf1_ref[...], dn,
                        preferred_element_type=jnp.float32)
    ap_ref[:, 0:128] = a + bf_ref[...]
    p = jnp.dot(xq, wge_ref[...], preferred_element_type=jnp.float32)
    ap_ref[:, 128:256] = p + bg_ref[...]
    bm = lax.dot_general(fea_s, wf2_ref[...], dn,
                         preferred_element_type=jnp.float32)
    t_ref[:, 0:128] = bm
    t_ref[:, 128:132] = xs
    t_ref[:, 132:256] = jnp.zeros_like(t_ref[:, 132:256])


def _prep(fea, fea_s, xyzT, xyz_sT, wf1, wf2, wge, bf2, bg2):
    b, c, n = fea.shape
    pb = 512
    nb = n // pb
    grid = (b, nb)
    out_c = wf1.shape[1]
    return pl.pallas_call(
        _prep_body,
        grid=grid,
        in_specs=[
            pl.BlockSpec((1, c, pb), lambda i, j: (i, 0, j)),
            pl.BlockSpec((1, c, pb), lambda i, j: (i, 0, j)),
            pl.BlockSpec((1, pb, 4), lambda i, j: (i, j, 0)),
            pl.BlockSpec((1, pb, 4), lambda i, j: (i, j, 0)),
            pl.BlockSpec((c, out_c), lambda i, j: (0, 0)),
            pl.BlockSpec((c, out_c), lambda i, j: (0, 0)),
            pl.BlockSpec((4, out_c), lambda i, j: (0, 0)),
            pl.BlockSpec((1, out_c), lambda i, j: (0, 0)),
            pl.BlockSpec((1, out_c), lambda i, j: (0, 0)),
        ],
        out_specs=[
            pl.BlockSpec((pb, TW), lambda i, j: (i * nb + j, 0)),
            pl.BlockSpec((pb, 2 * out_c), lambda i, j: (i * nb + j, 0)),
        ],
        out_shape=[
            jax.ShapeDtypeStruct((b * n, TW), jnp.float32),
            jax.ShapeDtypeStruct((b * n, 2 * out_c), jnp.float32),
        ],
    )(fea, fea_s, xyzT, xyz_sT, wf1, wf2, wge, bf2, bg2)


def _knn_body(xyzT_ref, xyz_s_ref, idx_ref, *, n, qb, k):
    bi = pl.program_id(0)
    xq = xyzT_ref[0]
    xs = xyz_s_ref[0]
    xqb = xq.astype(jnp.bfloat16).astype(jnp.float32)
    xsb = xs.astype(jnp.bfloat16).astype(jnp.float32)
    e = (xqb[:, 0:1] * xsb[0:1, :]
         + xqb[:, 1:2] * xsb[1:2, :]) + xqb[:, 2:3] * xsb[2:3, :]
    inner = -2.0 * e
    a2 = (xq[:, 0:1] * xq[:, 0:1] + xq[:, 1:2] * xq[:, 1:2]) \
        + xq[:, 2:3] * xq[:, 2:3]
    b2 = (xs[0:1, :] * xs[0:1, :] + xs[1:2, :] * xs[1:2, :]) \
        + xs[2:3, :] * xs[2:3, :]
    dis = (-a2 - inner) - b2
    iota = lax.broadcasted_iota(jnp.int32, (qb, n), 1)
    cols = []
    d = dis
    for _ in range(k):
        m = jnp.max(d, axis=1, keepdims=True)
        sel = jnp.min(jnp.where(d == m, iota, jnp.int32(n)),
                      axis=1, keepdims=True)
        cols.append(sel)
        d = jnp.where(iota == sel, NEG_INF, d)
    idx_ref[0] = jnp.concatenate(cols, axis=1) + bi * n


def _knn(xyzT, xyz_s):
    b, n, _ = xyzT.shape
    qb = 128
    grid = (b, n // qb)
    body = functools.partial(_knn_body, n=n, qb=qb, k=KNN)
    return pl.pallas_call(
        body,
        grid=grid,
        in_specs=[
            pl.BlockSpec((1, qb, 4), lambda i, j: (i, j, 0)),
            pl.BlockSpec((1, 3, n), lambda i, j: (i, 0, 0)),
        ],
        out_specs=pl.BlockSpec((1, qb, KNN), lambda i, j: (i, j, 0)),
        out_shape=jax.ShapeDtypeStruct((b, n, KNN), jnp.int32),
    )(xyzT, xyz_s)


def _gather_rows(table, idx):
    m = idx.shape[0]
    tw = table.shape[1]
    per_w = m // SC_NW
    ch = 128
    n_ch = per_w // ch
    mesh = plsc.VectorSubcoreMesh(core_axis_name="c", subcore_axis_name="s")

    @functools.partial(
        pl.kernel,
        out_type=jax.ShapeDtypeStruct((m, tw), jnp.float32),
        mesh=mesh,
        scratch_types=[
            pltpu.VMEM((ch,), jnp.int32),
            pltpu.VMEM((ch, tw), jnp.float32),
            pltpu.SemaphoreType.DMA,
        ],
    )
    def gather_k(t_hbm, idx_hbm, out_hbm, idx_v, rows_v, sem):
        wid = lax.axis_index("s") * SC_NC + lax.axis_index("c")
        base = wid * per_w

        def body(i, carry):
            off = base + i * ch
            pltpu.sync_copy(idx_hbm.at[pl.ds(off, ch)], idx_v)
            pltpu.async_copy(t_hbm.at[idx_v], rows_v, sem).wait()
            pltpu.sync_copy(rows_v, out_hbm.at[pl.ds(off, ch)])
            return carry

        lax.fori_loop(0, n_ch, body, 0)

    return gather_k(table, idx)


def _final_body(g_ref, ap_ref, xyzT_ref, w0_ref, wgs_ref, out_ref,
                *, fb, k, tw):
    flat = g_ref[...]
    q = jnp.dot(flat[:, 128:132], wgs_ref[...],
                preferred_element_type=jnp.float32)
    r = jnp.reshape(flat, (fb, k, tw))
    gf = r[:, :, 0:128]
    gx = r[:, :, 128:131]
    ap = ap_ref[...]
    a = ap[:, None, 0:128]
    p = ap[:, None, 128:256]
    xq = xyzT_ref[0][:, None, 0:3]
    diff = xq - gx
    d = jnp.sqrt(jnp.sum(diff * diff, axis=2, keepdims=True))
    f = jnp.maximum(a + gf, 0.0)
    g = jnp.maximum(p + jnp.reshape(q, (fb, k, 128))
                    + w0_ref[...][None] * d, 0.0)
    out_ref[0] = jnp.max(f * g, axis=1)


def _final(g, ap, xyzT, w0, wgs):
    b, n, _ = xyzT.shape
    out_c = w0.shape[1]
    fb = 256
    nb = n // fb
    body = functools.partial(_final_body, fb=fb, k=KNN, tw=TW)
    return pl.pallas_call(
        body,
        grid=(b, nb),
        in_specs=[
            pl.BlockSpec((fb * KNN, TW), lambda i, j: (i * nb + j, 0)),
            pl.BlockSpec((fb, 2 * out_c), lambda i, j: (i * nb + j, 0)),
            pl.BlockSpec((1, fb, 4), lambda i, j: (i, j, 0)),
            pl.BlockSpec((1, out_c), lambda i, j: (0, 0)),
            pl.BlockSpec((4, out_c), lambda i, j: (0, 0)),
        ],
        out_specs=pl.BlockSpec((1, fb, out_c), lambda i, j: (i, j, 0)),
        out_shape=jax.ShapeDtypeStruct((b, n, out_c), jnp.float32),
    )(g, ap, xyzT, w0, wgs)


def kernel(xyz, xyz_s, fea, fea_s, Wf, bf, Wg, bg):
    b, c, n = fea.shape
    out_c = Wf.shape[0]
    pad = jnp.zeros((b, n, 1), jnp.float32)
    xyzT = jnp.concatenate([jnp.swapaxes(xyz, 1, 2), pad], axis=2)
    xyz_sT = jnp.concatenate([jnp.swapaxes(xyz_s, 1, 2), pad], axis=2)
    wf1 = Wf[:, :c].T
    wf2 = Wf[:, c:].T
    wpad = jnp.zeros((1, out_c), jnp.float32)
    wge = jnp.concatenate([(Wg[:, 1:4] + Wg[:, 7:10]).T, wpad], axis=0)
    wgs = jnp.concatenate([(Wg[:, 4:7] - Wg[:, 7:10]).T, wpad], axis=0)
    w0 = Wg[:, 0:1].T
    bf2 = bf.reshape(1, out_c)
    bg2 = bg.reshape(1, out_c)

    table, ap = _prep(fea, fea_s, xyzT, xyz_sT, wf1, wf2, wge, bf2, bg2)
    idx = _knn(xyzT, xyz_s)
    gathered = _gather_rows(table, idx.reshape(-1))
    out = _final(gathered, ap, xyzT, w0, wgs)
    return jnp.swapaxes(out, 1, 2)
